# Initial kernel scaffold; baseline (speedup 1.0000x reference)
import jax
import jax.numpy as jnp
from jax.experimental import pallas as pl


def kernel(atom_features, bond_info, params):
    raise NotImplementedError("write your pallas kernel here")


# speedup vs baseline: 3.0165x; 3.0165x over previous
import functools

import jax
import jax.numpy as jnp
from jax import lax
from jax.experimental import pallas as pl
from jax.experimental.pallas import tpu as pltpu
from jax.experimental.pallas import tpu_sc as plsc

N_NODES = 10000
N_EDGES = 320000
NUM_BOND_TYPES = 4
BN_EPS = 1e-5

_NW = 32
_EDGE_B = 128
_K = 80
_EP = _NW * _K * _EDGE_B
_ACC_ROWS = 40960
_PAD_ROWS = _ACC_ROWS - N_NODES * NUM_BOND_TYPES
_ZROWS = 640
_ROWS_PER_TILE = _ACC_ROWS // 16


def _sc_agg(dch: int):
    mesh = plsc.VectorSubcoreMesh(core_axis_name="c", subcore_axis_name="s")

    @functools.partial(
        pl.kernel,
        out_type=jax.ShapeDtypeStruct((2, dch, _ACC_ROWS, 32), jnp.float32),
        mesh=mesh,
        compiler_params=pltpu.CompilerParams(use_tc_tiling_on_sc=False),
        scratch_types=[
            pltpu.VMEM((_K, _EDGE_B), jnp.int32),
            pltpu.VMEM((_K, _EDGE_B), jnp.int32),
            pltpu.VMEM((_EDGE_B, 32), jnp.float32),
            pltpu.VMEM((_ZROWS, 32), jnp.float32),
            pltpu.VMEM_SHARED((_ACC_ROWS, 32), jnp.float32),
            pltpu.SemaphoreType.DMA,
        ],
    )
    def agg(xflat, idxp, gidx, zc, out, idx_v, gidx_v, rows_v, zbuf, acc, sem):
        cid = lax.axis_index("c")
        ws = lax.axis_index("s")
        wid = cid * 16 + ws
        pltpu.sync_copy(zc, zbuf)
        pltpu.sync_copy(idxp.at[wid], idx_v)
        for c in range(dch):
            pltpu.sync_copy(gidx.at[c, wid], gidx_v)
            for q in range(4):
                pltpu.sync_copy(
                    zbuf, acc.at[pl.ds(ws * _ROWS_PER_TILE + q * _ZROWS, _ZROWS)])
            plsc.subcore_barrier()

            @pl.loop(0, _K)
            def _(j):
                pltpu.async_copy(xflat.at[gidx_v.at[j]], rows_v, sem).wait()
                pltpu.sync_copy(rows_v, acc.at[idx_v.at[j]], add=True)

            plsc.subcore_barrier()
            pltpu.sync_copy(
                acc.at[pl.ds(ws * _ROWS_PER_TILE, _ROWS_PER_TILE)],
                out.at[cid, c, pl.ds(ws * _ROWS_PER_TILE, _ROWS_PER_TILE), :])

    return agg


def _elu(z):
    return jnp.where(z > 0, z, jnp.exp(z) - 1.0)


_RB = 1000
_PREC = lax.Precision.HIGHEST


def _bn_linear(x, scale, beta, w):
    n, fin = x.shape
    h = w.shape[1]

    def body(x_ref, s_ref, b_ref, w_ref, o_ref):
        z = x_ref[...] * s_ref[...] + b_ref[...]
        o_ref[...] = jnp.dot(_elu(z), w_ref[...],
                             preferred_element_type=jnp.float32,
                             precision=_PREC)

    return pl.pallas_call(
        body,
        grid=(n // _RB,),
        in_specs=[
            pl.BlockSpec((_RB, fin), lambda i: (i, 0)),
            pl.BlockSpec((1, fin), lambda i: (0, 0)),
            pl.BlockSpec((1, fin), lambda i: (0, 0)),
            pl.BlockSpec((fin, h), lambda i: (0, 0)),
        ],
        out_specs=pl.BlockSpec((_RB, h), lambda i: (i, 0)),
        out_shape=jax.ShapeDtypeStruct((n, h), jnp.float32),
    )(x, scale.reshape(1, fin), beta.reshape(1, fin), w)


def _molconv_dense(x, aggp, gamma, beta, w):
    n, d = x.shape
    h = w.shape[1]
    scale = gamma / jnp.sqrt(jnp.float32(1.0 + BN_EPS))
    sp = jnp.concatenate([scale[:d].reshape(1, 1, d),
                          scale[d:].reshape(4, 1, d)], axis=0)
    bp = jnp.concatenate([beta[:d].reshape(1, 1, d),
                          beta[d:].reshape(4, 1, d)], axis=0)
    wp = jnp.concatenate([w[:d].reshape(1, d, h),
                          w[d:].reshape(4, d, h)], axis=0)

    dch = d // 32

    def body(x_ref, a_ref, s_ref, b_ref, w_ref, o_ref):
        z = x_ref[...] * s_ref[0] + b_ref[0]
        acc = jnp.dot(_elu(z), w_ref[0], preferred_element_type=jnp.float32,
                      precision=_PREC)
        a = a_ref[0] + a_ref[1]
        for t in range(4):
            zt = jnp.concatenate(
                [a[ch, :, t * 32:(t + 1) * 32] for ch in range(dch)], axis=-1)
            z = zt * s_ref[t + 1] + b_ref[t + 1]
            acc += jnp.dot(_elu(z), w_ref[t + 1],
                           preferred_element_type=jnp.float32, precision=_PREC)
        o_ref[...] = acc

    return pl.pallas_call(
        body,
        grid=(n // _RB,),
        in_specs=[
            pl.BlockSpec((_RB, d), lambda i: (i, 0)),
            pl.BlockSpec((2, dch, _RB, 128), lambda i: (0, 0, i, 0)),
            pl.BlockSpec((5, 1, d), lambda i: (0, 0, 0)),
            pl.BlockSpec((5, 1, d), lambda i: (0, 0, 0)),
            pl.BlockSpec((5, d, h), lambda i: (0, 0, 0)),
        ],
        out_specs=pl.BlockSpec((_RB, h), lambda i: (i, 0)),
        out_shape=jax.ShapeDtypeStruct((n, h), jnp.float32),
    )(x, aggp, sp, bp, wp)


def _edge_plans(bond_info):
    begin = bond_info[:, 0]
    end = bond_info[:, 1]
    bt = bond_info[:, 2] % NUM_BOND_TYPES
    idx = end * NUM_BOND_TYPES + bt
    pad = _EP - N_EDGES
    pr = jnp.arange(pad, dtype=jnp.int32)
    idxp = jnp.concatenate(
        [idx, N_NODES * NUM_BOND_TYPES + pr % _PAD_ROWS]).reshape(_NW, _K, _EDGE_B)
    beginp = jnp.concatenate([begin, pr % N_NODES])
    gidx = {}
    for dch in (4, 8):
        gidx[dch] = (
            beginp[None, :] * dch + jnp.arange(dch, dtype=jnp.int32)[:, None]
        ).reshape(dch, _NW, _K, _EDGE_B)
    zc = jnp.zeros((_ZROWS, 32), jnp.float32)
    return idxp, gidx, zc


def kernel(atom_features, bond_info, params):
    idxp, gidx, zc = _edge_plans(bond_info)

    def molconv(x, gamma, beta, w):
        d = x.shape[1]
        dch = d // 32
        xflat = x.reshape(N_NODES * dch, 32)
        aggp = _sc_agg(dch)(xflat, idxp, gidx[dch], zc)
        aggp = aggp.reshape(2, dch, _ACC_ROWS // 4, 128)
        return _molconv_dense(x, aggp, gamma, beta, w)

    def bn_scale(gamma):
        return gamma / jnp.sqrt(jnp.float32(1.0 + BN_EPS))

    x = atom_features
    for i in range(2):
        x = molconv(x, params['causal%d_gamma' % i], params['causal%d_beta' % i],
                    params['causal%d_W' % i])
    feats = [x]
    for i in range(3):
        cat = jnp.concatenate(feats, axis=-1)
        b = _bn_linear(cat, bn_scale(params['dense%d_bn_gamma' % i]),
                       params['dense%d_bn_beta' % i], params['dense%d_bn_W' % i])
        y = molconv(b, params['dense%d_conv_gamma' % i],
                    params['dense%d_conv_beta' % i], params['dense%d_conv_W' % i])
        feats.append(y)
    cat = jnp.concatenate(feats, axis=-1)
    return _bn_linear(cat, bn_scale(params['out_gamma']), params['out_beta'],
                      params['out_W'])


# speedup vs baseline: 4.8140x; 1.5959x over previous
import functools

import jax
import jax.numpy as jnp
from jax import lax
from jax.experimental import pallas as pl
from jax.experimental.pallas import tpu as pltpu
from jax.experimental.pallas import tpu_sc as plsc

N_NODES = 10000
N_EDGES = 320000
NUM_BOND_TYPES = 4
BN_EPS = 1e-5

_NW = 32
_EDGE_B = 128
_K = 80
_EP = _NW * _K * _EDGE_B
_NB = 2
_NSLOT = 2 * _NB
_ACC_ROWS = 40448
_PAD_ROWS = _ACC_ROWS - N_NODES * NUM_BOND_TYPES
_ZROWS = 158
_ROWS_PER_TILE = _ACC_ROWS // 16


def _sc_agg(dch: int):
    mesh = plsc.VectorSubcoreMesh(core_axis_name="c", subcore_axis_name="s")

    @functools.partial(
        pl.kernel,
        out_type=jax.ShapeDtypeStruct((2, dch, _ACC_ROWS, 32), jnp.float32),
        mesh=mesh,
        compiler_params=pltpu.CompilerParams(use_tc_tiling_on_sc=False),
        scratch_types=[
            pltpu.VMEM((_K, _EDGE_B), jnp.int32),
            pltpu.VMEM((_K + _NB, _EDGE_B), jnp.int32),
            pltpu.VMEM((_NSLOT, _EDGE_B, 32), jnp.float32),
            pltpu.VMEM((_ZROWS, 32), jnp.float32),
            pltpu.VMEM_SHARED((_ACC_ROWS, 32), jnp.float32),
            [pltpu.SemaphoreType.DMA] * _NSLOT,
            [pltpu.SemaphoreType.DMA] * _NSLOT,
        ],
    )
    def agg(xflat, idxp, gidx, zc, out, idx_v, gidx_v, rows_v, zbuf, acc,
            gsems, ssems):
        cid = lax.axis_index("c")
        ws = lax.axis_index("s")
        wid = cid * 16 + ws

        def gather(j, s):
            return pltpu.async_copy(xflat.at[gidx_v.at[j]], rows_v.at[s],
                                    gsems[s])

        def scatter(j, s):
            return pltpu.async_copy(rows_v.at[s], acc.at[idx_v.at[j]],
                                    ssems[s], add=True)

        pltpu.sync_copy(zc, zbuf)
        pltpu.sync_copy(idxp.at[wid], idx_v)
        for c in range(dch):
            pltpu.sync_copy(gidx.at[c, wid], gidx_v)
            for q in range(16):
                pltpu.sync_copy(
                    zbuf, acc.at[pl.ds(ws * _ROWS_PER_TILE + q * _ZROWS, _ZROWS)])
            plsc.subcore_barrier()

            @pl.loop(0, _K // _NSLOT)
            def _(g):
                base = g * _NSLOT
                gd = [gather(base + b, b) for b in range(_NSLOT)]
                sd = []
                for b in range(_NSLOT):
                    gd[b].wait()
                    sd.append(scatter(base + b, b))
                for b in range(_NSLOT):
                    sd[b].wait()

            plsc.subcore_barrier()
            pltpu.sync_copy(
                acc.at[pl.ds(ws * _ROWS_PER_TILE, _ROWS_PER_TILE)],
                out.at[cid, c, pl.ds(ws * _ROWS_PER_TILE, _ROWS_PER_TILE), :])

    return agg


def _elu(z):
    return jnp.where(z > 0, z, jnp.exp(z) - 1.0)


_RB = 1000
_PREC = lax.Precision.HIGHEST


def _bn_linear(x, scale, beta, w):
    n, fin = x.shape
    h = w.shape[1]

    def body(x_ref, s_ref, b_ref, w_ref, o_ref):
        z = x_ref[...] * s_ref[...] + b_ref[...]
        o_ref[...] = jnp.dot(_elu(z), w_ref[...],
                             preferred_element_type=jnp.float32,
                             precision=_PREC)

    return pl.pallas_call(
        body,
        grid=(n // _RB,),
        in_specs=[
            pl.BlockSpec((_RB, fin), lambda i: (i, 0)),
            pl.BlockSpec((1, fin), lambda i: (0, 0)),
            pl.BlockSpec((1, fin), lambda i: (0, 0)),
            pl.BlockSpec((fin, h), lambda i: (0, 0)),
        ],
        out_specs=pl.BlockSpec((_RB, h), lambda i: (i, 0)),
        out_shape=jax.ShapeDtypeStruct((n, h), jnp.float32),
    )(x, scale.reshape(1, fin), beta.reshape(1, fin), w)


def _molconv_dense(x, aggp, gamma, beta, w):
    n, d = x.shape
    h = w.shape[1]
    scale = gamma / jnp.sqrt(jnp.float32(1.0 + BN_EPS))
    sp = jnp.concatenate([scale[:d].reshape(1, 1, d),
                          scale[d:].reshape(4, 1, d)], axis=0)
    bp = jnp.concatenate([beta[:d].reshape(1, 1, d),
                          beta[d:].reshape(4, 1, d)], axis=0)
    wp = jnp.concatenate([w[:d].reshape(1, d, h),
                          w[d:].reshape(4, d, h)], axis=0)

    dch = d // 32

    def body(x_ref, a_ref, s_ref, b_ref, w_ref, o_ref):
        z = x_ref[...] * s_ref[0] + b_ref[0]
        acc = jnp.dot(_elu(z), w_ref[0], preferred_element_type=jnp.float32,
                      precision=_PREC)
        a = a_ref[0] + a_ref[1]
        for t in range(4):
            zt = jnp.concatenate(
                [a[ch, :, t * 32:(t + 1) * 32] for ch in range(dch)], axis=-1)
            z = zt * s_ref[t + 1] + b_ref[t + 1]
            acc += jnp.dot(_elu(z), w_ref[t + 1],
                           preferred_element_type=jnp.float32, precision=_PREC)
        o_ref[...] = acc

    return pl.pallas_call(
        body,
        grid=(n // _RB,),
        in_specs=[
            pl.BlockSpec((_RB, d), lambda i: (i, 0)),
            pl.BlockSpec((2, dch, _RB, 128), lambda i: (0, 0, i, 0)),
            pl.BlockSpec((5, 1, d), lambda i: (0, 0, 0)),
            pl.BlockSpec((5, 1, d), lambda i: (0, 0, 0)),
            pl.BlockSpec((5, d, h), lambda i: (0, 0, 0)),
        ],
        out_specs=pl.BlockSpec((_RB, h), lambda i: (i, 0)),
        out_shape=jax.ShapeDtypeStruct((n, h), jnp.float32),
    )(x, aggp, sp, bp, wp)


def _edge_plans(bond_info):
    begin = bond_info[:, 0]
    end = bond_info[:, 1]
    bt = bond_info[:, 2] % NUM_BOND_TYPES
    idx = end * NUM_BOND_TYPES + bt
    pad = _EP - N_EDGES
    pr = jnp.arange(pad, dtype=jnp.int32)
    idxp = jnp.concatenate(
        [idx, N_NODES * NUM_BOND_TYPES + pr % _PAD_ROWS]).reshape(_NW, _K, _EDGE_B)
    beginp = jnp.concatenate([begin, pr % N_NODES])
    gidx = {}
    for dch in (4, 8):
        g = (beginp[None, :] * dch + jnp.arange(dch, dtype=jnp.int32)[:, None]
             ).reshape(dch, _NW, _K, _EDGE_B)
        gidx[dch] = jnp.pad(g, ((0, 0), (0, 0), (0, _NB), (0, 0)))
    zc = jnp.zeros((_ZROWS, 32), jnp.float32)
    return idxp, gidx, zc


def kernel(atom_features, bond_info, params):
    idxp, gidx, zc = _edge_plans(bond_info)

    def molconv(x, gamma, beta, w):
        d = x.shape[1]
        dch = d // 32
        xflat = x.reshape(N_NODES * dch, 32)
        aggp = _sc_agg(dch)(xflat, idxp, gidx[dch], zc)
        aggp = aggp.reshape(2, dch, _ACC_ROWS // 4, 128)
        return _molconv_dense(x, aggp, gamma, beta, w)

    def bn_scale(gamma):
        return gamma / jnp.sqrt(jnp.float32(1.0 + BN_EPS))

    x = atom_features
    for i in range(2):
        x = molconv(x, params['causal%d_gamma' % i], params['causal%d_beta' % i],
                    params['causal%d_W' % i])
    feats = [x]
    for i in range(3):
        cat = jnp.concatenate(feats, axis=-1)
        b = _bn_linear(cat, bn_scale(params['dense%d_bn_gamma' % i]),
                       params['dense%d_bn_beta' % i], params['dense%d_bn_W' % i])
        y = molconv(b, params['dense%d_conv_gamma' % i],
                    params['dense%d_conv_beta' % i], params['dense%d_conv_W' % i])
        feats.append(y)
    cat = jnp.concatenate(feats, axis=-1)
    return _bn_linear(cat, bn_scale(params['out_gamma']), params['out_beta'],
                      params['out_W'])


# speedup vs baseline: 5.1517x; 1.0701x over previous
import functools

import jax
import jax.numpy as jnp
from jax import lax
from jax.experimental import pallas as pl
from jax.experimental.pallas import tpu as pltpu
from jax.experimental.pallas import tpu_sc as plsc

N_NODES = 10000
N_EDGES = 320000
NUM_BOND_TYPES = 4
BN_EPS = 1e-5

_NW = 32
_EDGE_B = 128
_K = 80
_EP = _NW * _K * _EDGE_B
_NSLOT = 5
_ACC_ROWS = 40448
_PAD_ROWS = _ACC_ROWS - N_NODES * NUM_BOND_TYPES
_ZROWS = 158
_ROWS_PER_TILE = _ACC_ROWS // 16


def _sc_agg(dch: int):
    mesh = plsc.VectorSubcoreMesh(core_axis_name="c", subcore_axis_name="s")

    @functools.partial(
        pl.kernel,
        out_type=jax.ShapeDtypeStruct((2, dch, _ACC_ROWS, 32), jnp.float32),
        mesh=mesh,
        compiler_params=pltpu.CompilerParams(use_tc_tiling_on_sc=False),
        scratch_types=[
            pltpu.VMEM((_K, _EDGE_B), jnp.int32),
            pltpu.VMEM((_K, _EDGE_B), jnp.int32),
            pltpu.VMEM((_NSLOT, _EDGE_B, 32), jnp.float32),
            pltpu.VMEM((_ZROWS, 32), jnp.float32),
            pltpu.VMEM_SHARED((_ACC_ROWS, 32), jnp.float32),
            [pltpu.SemaphoreType.DMA] * _NSLOT,
            [pltpu.SemaphoreType.DMA] * _NSLOT,
        ],
    )
    def agg(xflat, idxp, gidx, zc, out, idx_v, gidx_v, rows_v, zbuf, acc,
            gsems, ssems):
        cid = lax.axis_index("c")
        ws = lax.axis_index("s")
        wid = cid * 16 + ws

        def gather(j, s):
            return pltpu.async_copy(xflat.at[gidx_v.at[j]], rows_v.at[s],
                                    gsems[s])

        def scatter(j, s):
            return pltpu.async_copy(rows_v.at[s], acc.at[idx_v.at[j]],
                                    ssems[s], add=True)

        pltpu.sync_copy(zc, zbuf)
        pltpu.sync_copy(idxp.at[wid], idx_v)
        for c in range(dch):
            pltpu.sync_copy(gidx.at[c, wid], gidx_v)
            for q in range(16):
                pltpu.sync_copy(
                    zbuf, acc.at[pl.ds(ws * _ROWS_PER_TILE + q * _ZROWS, _ZROWS)])
            plsc.subcore_barrier()

            @pl.loop(0, _K // _NSLOT)
            def _(g):
                base = g * _NSLOT
                gd = [gather(base + b, b) for b in range(_NSLOT)]
                sd = []
                for b in range(_NSLOT):
                    gd[b].wait()
                    sd.append(scatter(base + b, b))
                for b in range(_NSLOT):
                    sd[b].wait()

            plsc.subcore_barrier()
            pltpu.sync_copy(
                acc.at[pl.ds(ws * _ROWS_PER_TILE, _ROWS_PER_TILE)],
                out.at[cid, c, pl.ds(ws * _ROWS_PER_TILE, _ROWS_PER_TILE), :])

    return agg


def _elu(z):
    return jnp.where(z > 0, z, jnp.exp(z) - 1.0)


_RB = 1000
_PREC = lax.Precision.HIGHEST


def _bn_linear(xs, gamma, beta, w):
    n = xs[0].shape[0]
    h = w.shape[1]
    fins = [x.shape[1] for x in xs]
    offs = [0]
    for f in fins:
        offs.append(offs[-1] + f)
    scale = gamma / jnp.sqrt(jnp.float32(1.0 + BN_EPS))
    sps = [scale[offs[k]:offs[k + 1]].reshape(1, fins[k]) for k in range(len(xs))]
    bps = [beta[offs[k]:offs[k + 1]].reshape(1, fins[k]) for k in range(len(xs))]
    wps = [w[offs[k]:offs[k + 1]] for k in range(len(xs))]

    def body(*refs):
        nseg = len(fins)
        x_refs = refs[:nseg]
        s_refs = refs[nseg:2 * nseg]
        b_refs = refs[2 * nseg:3 * nseg]
        w_refs = refs[3 * nseg:4 * nseg]
        o_ref = refs[4 * nseg]
        acc = None
        for k in range(nseg):
            z = x_refs[k][...] * s_refs[k][...] + b_refs[k][...]
            part = jnp.dot(_elu(z), w_refs[k][...],
                           preferred_element_type=jnp.float32, precision=_PREC)
            acc = part if acc is None else acc + part
        o_ref[...] = acc

    return pl.pallas_call(
        body,
        grid=(n // _RB,),
        in_specs=(
            [pl.BlockSpec((_RB, f), lambda i: (i, 0)) for f in fins]
            + [pl.BlockSpec((1, f), lambda i: (0, 0)) for f in fins]
            + [pl.BlockSpec((1, f), lambda i: (0, 0)) for f in fins]
            + [pl.BlockSpec((f, h), lambda i: (0, 0)) for f in fins]
        ),
        out_specs=pl.BlockSpec((_RB, h), lambda i: (i, 0)),
        out_shape=jax.ShapeDtypeStruct((n, h), jnp.float32),
    )(*xs, *sps, *bps, *wps)


def _molconv_dense(x, aggp, gamma, beta, w):
    n, d = x.shape
    h = w.shape[1]
    scale = gamma / jnp.sqrt(jnp.float32(1.0 + BN_EPS))
    sp = jnp.concatenate([scale[:d].reshape(1, 1, d),
                          scale[d:].reshape(4, 1, d)], axis=0)
    bp = jnp.concatenate([beta[:d].reshape(1, 1, d),
                          beta[d:].reshape(4, 1, d)], axis=0)
    wp = jnp.concatenate([w[:d].reshape(1, d, h),
                          w[d:].reshape(4, d, h)], axis=0)

    dch = d // 32

    def body(x_ref, a_ref, s_ref, b_ref, w_ref, o_ref):
        z = x_ref[...] * s_ref[0] + b_ref[0]
        acc = jnp.dot(_elu(z), w_ref[0], preferred_element_type=jnp.float32,
                      precision=_PREC)
        a = a_ref[0] + a_ref[1]
        for t in range(4):
            zt = jnp.concatenate(
                [a[ch, :, t * 32:(t + 1) * 32] for ch in range(dch)], axis=-1)
            z = zt * s_ref[t + 1] + b_ref[t + 1]
            acc += jnp.dot(_elu(z), w_ref[t + 1],
                           preferred_element_type=jnp.float32, precision=_PREC)
        o_ref[...] = acc

    return pl.pallas_call(
        body,
        grid=(n // _RB,),
        in_specs=[
            pl.BlockSpec((_RB, d), lambda i: (i, 0)),
            pl.BlockSpec((2, dch, _RB, 128), lambda i: (0, 0, i, 0)),
            pl.BlockSpec((5, 1, d), lambda i: (0, 0, 0)),
            pl.BlockSpec((5, 1, d), lambda i: (0, 0, 0)),
            pl.BlockSpec((5, d, h), lambda i: (0, 0, 0)),
        ],
        out_specs=pl.BlockSpec((_RB, h), lambda i: (i, 0)),
        out_shape=jax.ShapeDtypeStruct((n, h), jnp.float32),
    )(x, aggp, sp, bp, wp)


def _edge_plans(bond_info):
    begin = bond_info[:, 0]
    end = bond_info[:, 1]
    bt = bond_info[:, 2] % NUM_BOND_TYPES
    idx = end * NUM_BOND_TYPES + bt
    pad = _EP - N_EDGES
    pr = jnp.arange(pad, dtype=jnp.int32)
    idxp = jnp.concatenate(
        [idx, N_NODES * NUM_BOND_TYPES + pr % _PAD_ROWS]).reshape(_NW, _K, _EDGE_B)
    beginp = jnp.concatenate([begin, pr % N_NODES])
    gidx = {}
    for dch in (4, 8):
        gidx[dch] = (
            beginp[None, :] * dch + jnp.arange(dch, dtype=jnp.int32)[:, None]
        ).reshape(dch, _NW, _K, _EDGE_B)
    zc = jnp.zeros((_ZROWS, 32), jnp.float32)
    return idxp, gidx, zc


def kernel(atom_features, bond_info, params):
    idxp, gidx, zc = _edge_plans(bond_info)

    def molconv(x, gamma, beta, w):
        d = x.shape[1]
        dch = d // 32
        xflat = x.reshape(N_NODES * dch, 32)
        aggp = _sc_agg(dch)(xflat, idxp, gidx[dch], zc)
        aggp = aggp.reshape(2, dch, _ACC_ROWS // 4, 128)
        return _molconv_dense(x, aggp, gamma, beta, w)

    x = atom_features
    for i in range(2):
        x = molconv(x, params['causal%d_gamma' % i], params['causal%d_beta' % i],
                    params['causal%d_W' % i])
    feats = [x]
    for i in range(3):
        b = _bn_linear(feats, params['dense%d_bn_gamma' % i],
                       params['dense%d_bn_beta' % i], params['dense%d_bn_W' % i])
        y = molconv(b, params['dense%d_conv_gamma' % i],
                    params['dense%d_conv_beta' % i], params['dense%d_conv_W' % i])
        feats.append(y)
    return _bn_linear(feats, params['out_gamma'], params['out_beta'],
                      params['out_W'])


# speedup vs baseline: 5.3831x; 1.0449x over previous
import functools

import jax
import jax.numpy as jnp
from jax import lax
from jax.experimental import pallas as pl
from jax.experimental.pallas import tpu as pltpu
from jax.experimental.pallas import tpu_sc as plsc

N_NODES = 10000
N_EDGES = 320000
NUM_BOND_TYPES = 4
BN_EPS = 1e-5

_NW = 32
_EDGE_B = 128
_K = 80
_EP = _NW * _K * _EDGE_B
_NSLOT = 5
_ACC_ROWS = 40448
_PAD_ROWS = _ACC_ROWS - N_NODES * NUM_BOND_TYPES
_ZROWS = 158
_ROWS_PER_TILE = _ACC_ROWS // 16


def _sc_agg(dch: int):
    mesh = plsc.VectorSubcoreMesh(core_axis_name="c", subcore_axis_name="s")

    @functools.partial(
        pl.kernel,
        out_type=jax.ShapeDtypeStruct((2, dch, _ACC_ROWS, 32), jnp.float32),
        mesh=mesh,
        compiler_params=pltpu.CompilerParams(use_tc_tiling_on_sc=False),
        scratch_types=[
            pltpu.VMEM((_K, _EDGE_B), jnp.int32),
            pltpu.VMEM((_K, _EDGE_B), jnp.int32),
            pltpu.VMEM((_NSLOT, _EDGE_B, 32), jnp.float32),
            pltpu.VMEM((_ZROWS, 32), jnp.float32),
            pltpu.VMEM_SHARED((_ACC_ROWS, 32), jnp.float32),
            [pltpu.SemaphoreType.DMA] * _NSLOT,
            [pltpu.SemaphoreType.DMA] * _NSLOT,
            pltpu.SemaphoreType.DMA,
            pltpu.SemaphoreType.DMA,
            pltpu.SemaphoreType.DMA,
        ],
    )
    def agg(xflat, idxp, gidx, zc, out, idx_v, gidx_v, rows_v, zbuf, acc,
            gsems, ssems, zsem, osem, xsem):
        cid = lax.axis_index("c")
        ws = lax.axis_index("s")
        wid = cid * 16 + ws

        def gather(j, s):
            return pltpu.async_copy(xflat.at[gidx_v.at[j]], rows_v.at[s],
                                    gsems[s])

        def scatter(j, s):
            return pltpu.async_copy(rows_v.at[s], acc.at[idx_v.at[j]],
                                    ssems[s], add=True)

        def issue_zero():
            return [pltpu.async_copy(
                zbuf, acc.at[pl.ds(ws * _ROWS_PER_TILE + q * _ZROWS, _ZROWS)],
                zsem) for q in range(_ROWS_PER_TILE // _ZROWS)]

        pltpu.sync_copy(zc, zbuf)
        pltpu.sync_copy(idxp.at[wid], idx_v)
        pltpu.sync_copy(gidx.at[0, wid], gidx_v)
        zds = issue_zero()
        for c in range(dch):
            gd0 = [gather(b, b) for b in range(_NSLOT)]
            for zd in zds:
                zd.wait()
            plsc.subcore_barrier()
            sd0 = []
            for b in range(_NSLOT):
                gd0[b].wait()
                sd0.append(scatter(b, b))
            for b in range(_NSLOT):
                sd0[b].wait()

            @pl.loop(1, _K // _NSLOT)
            def _(g):
                base = g * _NSLOT
                gd = [gather(base + b, b) for b in range(_NSLOT)]
                sd = []
                for b in range(_NSLOT):
                    gd[b].wait()
                    sd.append(scatter(base + b, b))
                for b in range(_NSLOT):
                    sd[b].wait()

            plsc.subcore_barrier()
            od = pltpu.async_copy(
                acc.at[pl.ds(ws * _ROWS_PER_TILE, _ROWS_PER_TILE)],
                out.at[cid, c, pl.ds(ws * _ROWS_PER_TILE, _ROWS_PER_TILE), :],
                osem)
            if c + 1 < dch:
                gl = pltpu.async_copy(gidx.at[c + 1, wid], gidx_v, xsem)
                od.wait()
                zds = issue_zero()
                gl.wait()
            else:
                od.wait()

    return agg


def _elu(z):
    return jnp.where(z > 0, z, jnp.exp(z) - 1.0)


_RB = 1000
_PREC = lax.Precision.HIGHEST


def _bn_linear(xs, gamma, beta, w):
    n = xs[0].shape[0]
    h = w.shape[1]
    fins = [x.shape[1] for x in xs]
    offs = [0]
    for f in fins:
        offs.append(offs[-1] + f)
    scale = gamma / jnp.sqrt(jnp.float32(1.0 + BN_EPS))
    sps = [scale[offs[k]:offs[k + 1]].reshape(1, fins[k]) for k in range(len(xs))]
    bps = [beta[offs[k]:offs[k + 1]].reshape(1, fins[k]) for k in range(len(xs))]
    wps = [w[offs[k]:offs[k + 1]] for k in range(len(xs))]

    def body(*refs):
        nseg = len(fins)
        x_refs = refs[:nseg]
        s_refs = refs[nseg:2 * nseg]
        b_refs = refs[2 * nseg:3 * nseg]
        w_refs = refs[3 * nseg:4 * nseg]
        o_ref = refs[4 * nseg]
        acc = None
        for k in range(nseg):
            z = x_refs[k][...] * s_refs[k][...] + b_refs[k][...]
            part = jnp.dot(_elu(z), w_refs[k][...],
                           preferred_element_type=jnp.float32, precision=_PREC)
            acc = part if acc is None else acc + part
        o_ref[...] = acc

    return pl.pallas_call(
        body,
        grid=(n // _RB,),
        in_specs=(
            [pl.BlockSpec((_RB, f), lambda i: (i, 0)) for f in fins]
            + [pl.BlockSpec((1, f), lambda i: (0, 0)) for f in fins]
            + [pl.BlockSpec((1, f), lambda i: (0, 0)) for f in fins]
            + [pl.BlockSpec((f, h), lambda i: (0, 0)) for f in fins]
        ),
        out_specs=pl.BlockSpec((_RB, h), lambda i: (i, 0)),
        out_shape=jax.ShapeDtypeStruct((n, h), jnp.float32),
    )(*xs, *sps, *bps, *wps)


def _molconv_dense(x, aggp, gamma, beta, w):
    n, d = x.shape
    h = w.shape[1]
    scale = gamma / jnp.sqrt(jnp.float32(1.0 + BN_EPS))
    sp = jnp.concatenate([scale[:d].reshape(1, 1, d),
                          scale[d:].reshape(4, 1, d)], axis=0)
    bp = jnp.concatenate([beta[:d].reshape(1, 1, d),
                          beta[d:].reshape(4, 1, d)], axis=0)
    wp = jnp.concatenate([w[:d].reshape(1, d, h),
                          w[d:].reshape(4, d, h)], axis=0)

    dch = d // 32

    def body(x_ref, a_ref, s_ref, b_ref, w_ref, o_ref):
        z = x_ref[...] * s_ref[0] + b_ref[0]
        acc = jnp.dot(_elu(z), w_ref[0], preferred_element_type=jnp.float32,
                      precision=_PREC)
        a = a_ref[0] + a_ref[1]
        for t in range(4):
            zt = jnp.concatenate(
                [a[ch, :, t * 32:(t + 1) * 32] for ch in range(dch)], axis=-1)
            z = zt * s_ref[t + 1] + b_ref[t + 1]
            acc += jnp.dot(_elu(z), w_ref[t + 1],
                           preferred_element_type=jnp.float32, precision=_PREC)
        o_ref[...] = acc

    return pl.pallas_call(
        body,
        grid=(n // _RB,),
        in_specs=[
            pl.BlockSpec((_RB, d), lambda i: (i, 0)),
            pl.BlockSpec((2, dch, _RB, 128), lambda i: (0, 0, i, 0)),
            pl.BlockSpec((5, 1, d), lambda i: (0, 0, 0)),
            pl.BlockSpec((5, 1, d), lambda i: (0, 0, 0)),
            pl.BlockSpec((5, d, h), lambda i: (0, 0, 0)),
        ],
        out_specs=pl.BlockSpec((_RB, h), lambda i: (i, 0)),
        out_shape=jax.ShapeDtypeStruct((n, h), jnp.float32),
    )(x, aggp, sp, bp, wp)


def _edge_plans(bond_info):
    begin = bond_info[:, 0]
    end = bond_info[:, 1]
    bt = bond_info[:, 2] % NUM_BOND_TYPES
    idx = end * NUM_BOND_TYPES + bt
    pad = _EP - N_EDGES
    pr = jnp.arange(pad, dtype=jnp.int32)
    idxp = jnp.concatenate(
        [idx, N_NODES * NUM_BOND_TYPES + pr % _PAD_ROWS]).reshape(_NW, _K, _EDGE_B)
    beginp = jnp.concatenate([begin, pr % N_NODES])
    gidx = {}
    for dch in (4, 8):
        gidx[dch] = (
            beginp[None, :] * dch + jnp.arange(dch, dtype=jnp.int32)[:, None]
        ).reshape(dch, _NW, _K, _EDGE_B)
    zc = jnp.zeros((_ZROWS, 32), jnp.float32)
    return idxp, gidx, zc


def kernel(atom_features, bond_info, params):
    idxp, gidx, zc = _edge_plans(bond_info)

    def molconv(x, gamma, beta, w):
        d = x.shape[1]
        dch = d // 32
        xflat = x.reshape(N_NODES * dch, 32)
        aggp = _sc_agg(dch)(xflat, idxp, gidx[dch], zc)
        aggp = aggp.reshape(2, dch, _ACC_ROWS // 4, 128)
        return _molconv_dense(x, aggp, gamma, beta, w)

    x = atom_features
    for i in range(2):
        x = molconv(x, params['causal%d_gamma' % i], params['causal%d_beta' % i],
                    params['causal%d_W' % i])
    feats = [x]
    for i in range(3):
        b = _bn_linear(feats, params['dense%d_bn_gamma' % i],
                       params['dense%d_bn_beta' % i], params['dense%d_bn_W' % i])
        y = molconv(b, params['dense%d_conv_gamma' % i],
                    params['dense%d_conv_beta' % i], params['dense%d_conv_W' % i])
        feats.append(y)
    return _bn_linear(feats, params['out_gamma'], params['out_beta'],
                      params['out_W'])


# speedup vs baseline: 6.5952x; 1.2252x over previous
import functools

import jax
import jax.numpy as jnp
from jax import lax
from jax.experimental import pallas as pl
from jax.experimental.pallas import tpu as pltpu
from jax.experimental.pallas import tpu_sc as plsc

N_NODES = 10000
N_EDGES = 320000
NUM_BOND_TYPES = 4
BN_EPS = 1e-5

_NW = 32
_EDGE_B = 128
_K = 80
_EP = _NW * _K * _EDGE_B
_NSLOT = 5
_L = 3
_ACC_ROWS = 40448
_PAD_ROWS = _ACC_ROWS - N_NODES * NUM_BOND_TYPES
_ZROWS = 158
_ROWS_PER_TILE = _ACC_ROWS // 16


def _sc_agg(dch: int):
    mesh = plsc.VectorSubcoreMesh(core_axis_name="c", subcore_axis_name="s")

    @functools.partial(
        pl.kernel,
        out_type=jax.ShapeDtypeStruct((2, dch, _ACC_ROWS, 32), jnp.float32),
        mesh=mesh,
        compiler_params=pltpu.CompilerParams(use_tc_tiling_on_sc=False),
        scratch_types=[
            pltpu.VMEM((_K, _EDGE_B), jnp.int32),
            pltpu.VMEM((_K, _EDGE_B), jnp.int32),
            pltpu.VMEM((_NSLOT, _EDGE_B, 32), jnp.float32),
            pltpu.VMEM((_ZROWS, 32), jnp.float32),
            pltpu.VMEM_SHARED((_ACC_ROWS, 32), jnp.float32),
            [pltpu.SemaphoreType.DMA] * _NSLOT,
            [pltpu.SemaphoreType.DMA] * _NSLOT,
            pltpu.SemaphoreType.DMA,
            pltpu.SemaphoreType.DMA,
            pltpu.SemaphoreType.DMA,
        ],
    )
    def agg(xflat, idxp, gidx, zc, out, idx_v, gidx_v, rows_v, zbuf, acc,
            gsems, ssems, zsem, osem, xsem):
        cid = lax.axis_index("c")
        ws = lax.axis_index("s")
        wid = cid * 16 + ws

        def gather(j, s):
            return pltpu.async_copy(xflat.at[gidx_v.at[j]], rows_v.at[s],
                                    gsems[s])

        def gather_wait(j, s):
            pltpu.make_async_copy(xflat.at[gidx_v.at[j]], rows_v.at[s],
                                  gsems[s]).wait()

        def scatter(j, s):
            return pltpu.async_copy(rows_v.at[s], acc.at[idx_v.at[j]],
                                    ssems[s], add=True)

        def scatter_wait(j, s):
            pltpu.make_async_copy(rows_v.at[s], acc.at[idx_v.at[j]],
                                  ssems[s]).wait()

        def issue_zero():
            return [pltpu.async_copy(
                zbuf, acc.at[pl.ds(ws * _ROWS_PER_TILE + q * _ZROWS, _ZROWS)],
                zsem) for q in range(_ROWS_PER_TILE // _ZROWS)]

        pltpu.sync_copy(zc, zbuf)
        pltpu.sync_copy(idxp.at[wid], idx_v)
        pltpu.sync_copy(gidx.at[0, wid], gidx_v)
        zds = issue_zero()
        for c in range(dch):
            for b in range(_L):
                gather(b, b)
            for zd in zds:
                zd.wait()
            plsc.subcore_barrier()

            def stage(j):
                sw = j + _L - _NSLOT
                if sw >= 0:
                    scatter_wait(sw, sw % _NSLOT)
                if j + _L < _K:
                    gather(j + _L, (j + _L) % _NSLOT)
                gather_wait(j, j % _NSLOT)
                scatter(j, j % _NSLOT)

            for j in range(_NSLOT):
                stage(j)

            @pl.loop(1, _K // _NSLOT - 1)
            def _(g):
                for b in range(_NSLOT):
                    j = g * _NSLOT + b
                    s_la = (b + _L) % _NSLOT
                    scatter_wait(j + _L - _NSLOT, s_la)
                    gather(j + _L, s_la)
                    gather_wait(j, b)
                    scatter(j, b)

            for j in range(_K - _NSLOT, _K):
                stage(j)
            for j in range(_K - (_NSLOT - _L), _K):
                scatter_wait(j, j % _NSLOT)

            plsc.subcore_barrier()
            od = pltpu.async_copy(
                acc.at[pl.ds(ws * _ROWS_PER_TILE, _ROWS_PER_TILE)],
                out.at[cid, c, pl.ds(ws * _ROWS_PER_TILE, _ROWS_PER_TILE), :],
                osem)
            if c + 1 < dch:
                gl = pltpu.async_copy(gidx.at[c + 1, wid], gidx_v, xsem)
                od.wait()
                zds = issue_zero()
                gl.wait()
            else:
                od.wait()

    return agg


def _elu(z):
    return jnp.where(z > 0, z, jnp.exp(z) - 1.0)


_RB = 1000
_PREC = lax.Precision.HIGHEST


def _bn_linear(xs, gamma, beta, w):
    n = xs[0].shape[0]
    h = w.shape[1]
    fins = [x.shape[1] for x in xs]
    offs = [0]
    for f in fins:
        offs.append(offs[-1] + f)
    scale = gamma / jnp.sqrt(jnp.float32(1.0 + BN_EPS))
    sps = [scale[offs[k]:offs[k + 1]].reshape(1, fins[k]) for k in range(len(xs))]
    bps = [beta[offs[k]:offs[k + 1]].reshape(1, fins[k]) for k in range(len(xs))]
    wps = [w[offs[k]:offs[k + 1]] for k in range(len(xs))]

    def body(*refs):
        nseg = len(fins)
        x_refs = refs[:nseg]
        s_refs = refs[nseg:2 * nseg]
        b_refs = refs[2 * nseg:3 * nseg]
        w_refs = refs[3 * nseg:4 * nseg]
        o_ref = refs[4 * nseg]
        acc = None
        for k in range(nseg):
            z = x_refs[k][...] * s_refs[k][...] + b_refs[k][...]
            part = jnp.dot(_elu(z), w_refs[k][...],
                           preferred_element_type=jnp.float32, precision=_PREC)
            acc = part if acc is None else acc + part
        o_ref[...] = acc

    return pl.pallas_call(
        body,
        grid=(n // _RB,),
        in_specs=(
            [pl.BlockSpec((_RB, f), lambda i: (i, 0)) for f in fins]
            + [pl.BlockSpec((1, f), lambda i: (0, 0)) for f in fins]
            + [pl.BlockSpec((1, f), lambda i: (0, 0)) for f in fins]
            + [pl.BlockSpec((f, h), lambda i: (0, 0)) for f in fins]
        ),
        out_specs=pl.BlockSpec((_RB, h), lambda i: (i, 0)),
        out_shape=jax.ShapeDtypeStruct((n, h), jnp.float32),
    )(*xs, *sps, *bps, *wps)


def _molconv_dense(x, aggp, gamma, beta, w):
    n, d = x.shape
    h = w.shape[1]
    scale = gamma / jnp.sqrt(jnp.float32(1.0 + BN_EPS))
    sp = jnp.concatenate([scale[:d].reshape(1, 1, d),
                          scale[d:].reshape(4, 1, d)], axis=0)
    bp = jnp.concatenate([beta[:d].reshape(1, 1, d),
                          beta[d:].reshape(4, 1, d)], axis=0)
    wp = jnp.concatenate([w[:d].reshape(1, d, h),
                          w[d:].reshape(4, d, h)], axis=0)

    dch = d // 32

    def body(x_ref, a_ref, s_ref, b_ref, w_ref, o_ref):
        z = x_ref[...] * s_ref[0] + b_ref[0]
        acc = jnp.dot(_elu(z), w_ref[0], preferred_element_type=jnp.float32,
                      precision=_PREC)
        a = a_ref[0] + a_ref[1]
        for t in range(4):
            zt = jnp.concatenate(
                [a[ch, :, t * 32:(t + 1) * 32] for ch in range(dch)], axis=-1)
            z = zt * s_ref[t + 1] + b_ref[t + 1]
            acc += jnp.dot(_elu(z), w_ref[t + 1],
                           preferred_element_type=jnp.float32, precision=_PREC)
        o_ref[...] = acc

    return pl.pallas_call(
        body,
        grid=(n // _RB,),
        in_specs=[
            pl.BlockSpec((_RB, d), lambda i: (i, 0)),
            pl.BlockSpec((2, dch, _RB, 128), lambda i: (0, 0, i, 0)),
            pl.BlockSpec((5, 1, d), lambda i: (0, 0, 0)),
            pl.BlockSpec((5, 1, d), lambda i: (0, 0, 0)),
            pl.BlockSpec((5, d, h), lambda i: (0, 0, 0)),
        ],
        out_specs=pl.BlockSpec((_RB, h), lambda i: (i, 0)),
        out_shape=jax.ShapeDtypeStruct((n, h), jnp.float32),
    )(x, aggp, sp, bp, wp)


def _edge_plans(bond_info):
    begin = bond_info[:, 0]
    end = bond_info[:, 1]
    bt = bond_info[:, 2] % NUM_BOND_TYPES
    idx = end * NUM_BOND_TYPES + bt
    pad = _EP - N_EDGES
    pr = jnp.arange(pad, dtype=jnp.int32)
    idxp = jnp.concatenate(
        [idx, N_NODES * NUM_BOND_TYPES + pr % _PAD_ROWS]).reshape(_NW, _K, _EDGE_B)
    beginp = jnp.concatenate([begin, pr % N_NODES])
    gidx = {}
    for dch in (4, 8):
        gidx[dch] = (
            beginp[None, :] * dch + jnp.arange(dch, dtype=jnp.int32)[:, None]
        ).reshape(dch, _NW, _K, _EDGE_B)
    zc = jnp.zeros((_ZROWS, 32), jnp.float32)
    return idxp, gidx, zc


def kernel(atom_features, bond_info, params):
    idxp, gidx, zc = _edge_plans(bond_info)

    def molconv(x, gamma, beta, w):
        d = x.shape[1]
        dch = d // 32
        xflat = x.reshape(N_NODES * dch, 32)
        aggp = _sc_agg(dch)(xflat, idxp, gidx[dch], zc)
        aggp = aggp.reshape(2, dch, _ACC_ROWS // 4, 128)
        return _molconv_dense(x, aggp, gamma, beta, w)

    x = atom_features
    for i in range(2):
        x = molconv(x, params['causal%d_gamma' % i], params['causal%d_beta' % i],
                    params['causal%d_W' % i])
    feats = [x]
    for i in range(3):
        b = _bn_linear(feats, params['dense%d_bn_gamma' % i],
                       params['dense%d_bn_beta' % i], params['dense%d_bn_W' % i])
        y = molconv(b, params['dense%d_conv_gamma' % i],
                    params['dense%d_conv_beta' % i], params['dense%d_conv_W' % i])
        feats.append(y)
    return _bn_linear(feats, params['out_gamma'], params['out_beta'],
                      params['out_W'])


# speedup vs baseline: 6.8286x; 1.0354x over previous
import functools

import jax
import jax.numpy as jnp
from jax import lax
from jax.experimental import pallas as pl
from jax.experimental.pallas import tpu as pltpu
from jax.experimental.pallas import tpu_sc as plsc

N_NODES = 10000
N_EDGES = 320000
NUM_BOND_TYPES = 4
BN_EPS = 1e-5

_NW = 32
_EDGE_B = 128
_K = 80
_EP = _NW * _K * _EDGE_B
_NSLOT = 5
_L = 4
_ACC_ROWS = 40448
_PAD_ROWS = _ACC_ROWS - N_NODES * NUM_BOND_TYPES
_ZROWS = 158
_ROWS_PER_TILE = _ACC_ROWS // 16


def _sc_agg(dch: int):
    mesh = plsc.VectorSubcoreMesh(core_axis_name="c", subcore_axis_name="s")

    @functools.partial(
        pl.kernel,
        out_type=jax.ShapeDtypeStruct((2, dch, _ACC_ROWS, 32), jnp.float32),
        mesh=mesh,
        compiler_params=pltpu.CompilerParams(use_tc_tiling_on_sc=False),
        scratch_types=[
            pltpu.VMEM((_K, _EDGE_B), jnp.int32),
            pltpu.VMEM((_K, _EDGE_B), jnp.int32),
            pltpu.VMEM((_NSLOT, _EDGE_B, 32), jnp.float32),
            pltpu.VMEM((_ZROWS, 32), jnp.float32),
            pltpu.VMEM_SHARED((_ACC_ROWS, 32), jnp.float32),
            [pltpu.SemaphoreType.DMA] * _NSLOT,
            [pltpu.SemaphoreType.DMA] * _NSLOT,
            pltpu.SemaphoreType.DMA,
            pltpu.SemaphoreType.DMA,
            pltpu.SemaphoreType.DMA,
        ],
    )
    def agg(xflat, idxp, gidx, zc, out, idx_v, gidx_v, rows_v, zbuf, acc,
            gsems, ssems, zsem, osem, xsem):
        cid = lax.axis_index("c")
        ws = lax.axis_index("s")
        wid = cid * 16 + ws

        def gather(j, s):
            return pltpu.async_copy(xflat.at[gidx_v.at[j]], rows_v.at[s],
                                    gsems[s])

        def gather_wait(j, s):
            pltpu.make_async_copy(xflat.at[gidx_v.at[j]], rows_v.at[s],
                                  gsems[s]).wait()

        def scatter(j, s):
            return pltpu.async_copy(rows_v.at[s], acc.at[idx_v.at[j]],
                                    ssems[s], add=True)

        def scatter_wait(j, s):
            pltpu.make_async_copy(rows_v.at[s], acc.at[idx_v.at[j]],
                                  ssems[s]).wait()

        def issue_zero():
            return [pltpu.async_copy(
                zbuf, acc.at[pl.ds(ws * _ROWS_PER_TILE + q * _ZROWS, _ZROWS)],
                zsem) for q in range(_ROWS_PER_TILE // _ZROWS)]

        pltpu.sync_copy(zc, zbuf)
        pltpu.sync_copy(idxp.at[wid], idx_v)
        pltpu.sync_copy(gidx.at[0, wid], gidx_v)
        zds = issue_zero()
        for c in range(dch):
            for b in range(_L):
                gather(b, b)
            for zd in zds:
                zd.wait()
            plsc.subcore_barrier()

            def stage(j):
                sw = j + _L - _NSLOT
                if sw >= 0:
                    scatter_wait(sw, sw % _NSLOT)
                if j + _L < _K:
                    gather(j + _L, (j + _L) % _NSLOT)
                gather_wait(j, j % _NSLOT)
                scatter(j, j % _NSLOT)

            for j in range(_NSLOT):
                stage(j)

            @pl.loop(1, _K // _NSLOT - 1)
            def _(g):
                for b in range(_NSLOT):
                    j = g * _NSLOT + b
                    s_la = (b + _L) % _NSLOT
                    scatter_wait(j + _L - _NSLOT, s_la)
                    gather(j + _L, s_la)
                    gather_wait(j, b)
                    scatter(j, b)

            for j in range(_K - _NSLOT, _K):
                stage(j)
            for j in range(_K - (_NSLOT - _L), _K):
                scatter_wait(j, j % _NSLOT)

            plsc.subcore_barrier()
            od = pltpu.async_copy(
                acc.at[pl.ds(ws * _ROWS_PER_TILE, _ROWS_PER_TILE)],
                out.at[cid, c, pl.ds(ws * _ROWS_PER_TILE, _ROWS_PER_TILE), :],
                osem)
            if c + 1 < dch:
                gl = pltpu.async_copy(gidx.at[c + 1, wid], gidx_v, xsem)
                od.wait()
                zds = issue_zero()
                gl.wait()
            else:
                od.wait()

    return agg


def _elu(z):
    return jnp.where(z > 0, z, jnp.exp(z) - 1.0)


_RB = 1000
_PREC = lax.Precision.HIGHEST


def _bn_linear(xs, gamma, beta, w):
    n = xs[0].shape[0]
    h = w.shape[1]
    fins = [x.shape[1] for x in xs]
    offs = [0]
    for f in fins:
        offs.append(offs[-1] + f)
    scale = gamma / jnp.sqrt(jnp.float32(1.0 + BN_EPS))
    sps = [scale[offs[k]:offs[k + 1]].reshape(1, fins[k]) for k in range(len(xs))]
    bps = [beta[offs[k]:offs[k + 1]].reshape(1, fins[k]) for k in range(len(xs))]
    wps = [w[offs[k]:offs[k + 1]] for k in range(len(xs))]

    def body(*refs):
        nseg = len(fins)
        x_refs = refs[:nseg]
        s_refs = refs[nseg:2 * nseg]
        b_refs = refs[2 * nseg:3 * nseg]
        w_refs = refs[3 * nseg:4 * nseg]
        o_ref = refs[4 * nseg]
        acc = None
        for k in range(nseg):
            z = x_refs[k][...] * s_refs[k][...] + b_refs[k][...]
            part = jnp.dot(_elu(z), w_refs[k][...],
                           preferred_element_type=jnp.float32, precision=_PREC)
            acc = part if acc is None else acc + part
        o_ref[...] = acc

    return pl.pallas_call(
        body,
        grid=(n // _RB,),
        in_specs=(
            [pl.BlockSpec((_RB, f), lambda i: (i, 0)) for f in fins]
            + [pl.BlockSpec((1, f), lambda i: (0, 0)) for f in fins]
            + [pl.BlockSpec((1, f), lambda i: (0, 0)) for f in fins]
            + [pl.BlockSpec((f, h), lambda i: (0, 0)) for f in fins]
        ),
        out_specs=pl.BlockSpec((_RB, h), lambda i: (i, 0)),
        out_shape=jax.ShapeDtypeStruct((n, h), jnp.float32),
    )(*xs, *sps, *bps, *wps)


def _molconv_dense(x, aggp, gamma, beta, w):
    n, d = x.shape
    h = w.shape[1]
    scale = gamma / jnp.sqrt(jnp.float32(1.0 + BN_EPS))
    sp = jnp.concatenate([scale[:d].reshape(1, 1, d),
                          scale[d:].reshape(4, 1, d)], axis=0)
    bp = jnp.concatenate([beta[:d].reshape(1, 1, d),
                          beta[d:].reshape(4, 1, d)], axis=0)
    wp = jnp.concatenate([w[:d].reshape(1, d, h),
                          w[d:].reshape(4, d, h)], axis=0)

    dch = d // 32

    def body(x_ref, a_ref, s_ref, b_ref, w_ref, o_ref):
        z = x_ref[...] * s_ref[0] + b_ref[0]
        acc = jnp.dot(_elu(z), w_ref[0], preferred_element_type=jnp.float32,
                      precision=_PREC)
        a = a_ref[0] + a_ref[1]
        for t in range(4):
            zt = jnp.concatenate(
                [a[ch, :, t * 32:(t + 1) * 32] for ch in range(dch)], axis=-1)
            z = zt * s_ref[t + 1] + b_ref[t + 1]
            acc += jnp.dot(_elu(z), w_ref[t + 1],
                           preferred_element_type=jnp.float32, precision=_PREC)
        o_ref[...] = acc

    return pl.pallas_call(
        body,
        grid=(n // _RB,),
        in_specs=[
            pl.BlockSpec((_RB, d), lambda i: (i, 0)),
            pl.BlockSpec((2, dch, _RB, 128), lambda i: (0, 0, i, 0)),
            pl.BlockSpec((5, 1, d), lambda i: (0, 0, 0)),
            pl.BlockSpec((5, 1, d), lambda i: (0, 0, 0)),
            pl.BlockSpec((5, d, h), lambda i: (0, 0, 0)),
        ],
        out_specs=pl.BlockSpec((_RB, h), lambda i: (i, 0)),
        out_shape=jax.ShapeDtypeStruct((n, h), jnp.float32),
    )(x, aggp, sp, bp, wp)


def _edge_plans(bond_info):
    begin = bond_info[:, 0]
    end = bond_info[:, 1]
    bt = bond_info[:, 2] % NUM_BOND_TYPES
    idx = end * NUM_BOND_TYPES + bt
    pad = _EP - N_EDGES
    pr = jnp.arange(pad, dtype=jnp.int32)
    idxp = jnp.concatenate(
        [idx, N_NODES * NUM_BOND_TYPES + pr % _PAD_ROWS]).reshape(_NW, _K, _EDGE_B)
    beginp = jnp.concatenate([begin, pr % N_NODES])
    gidx = {}
    for dch in (4, 8):
        gidx[dch] = (
            beginp[None, :] * dch + jnp.arange(dch, dtype=jnp.int32)[:, None]
        ).reshape(dch, _NW, _K, _EDGE_B)
    zc = jnp.zeros((_ZROWS, 32), jnp.float32)
    return idxp, gidx, zc


def kernel(atom_features, bond_info, params):
    idxp, gidx, zc = _edge_plans(bond_info)

    def molconv(x, gamma, beta, w):
        d = x.shape[1]
        dch = d // 32
        xflat = x.reshape(N_NODES * dch, 32)
        aggp = _sc_agg(dch)(xflat, idxp, gidx[dch], zc)
        aggp = aggp.reshape(2, dch, _ACC_ROWS // 4, 128)
        return _molconv_dense(x, aggp, gamma, beta, w)

    x = atom_features
    for i in range(2):
        x = molconv(x, params['causal%d_gamma' % i], params['causal%d_beta' % i],
                    params['causal%d_W' % i])
    feats = [x]
    for i in range(3):
        b = _bn_linear(feats, params['dense%d_bn_gamma' % i],
                       params['dense%d_bn_beta' % i], params['dense%d_bn_W' % i])
        y = molconv(b, params['dense%d_conv_gamma' % i],
                    params['dense%d_conv_beta' % i], params['dense%d_conv_W' % i])
        feats.append(y)
    return _bn_linear(feats, params['out_gamma'], params['out_beta'],
                      params['out_W'])


# speedup vs baseline: 8.0162x; 1.1739x over previous
import functools

import jax
import jax.numpy as jnp
from jax import lax
from jax.experimental import pallas as pl
from jax.experimental.pallas import tpu as pltpu
from jax.experimental.pallas import tpu_sc as plsc

N_NODES = 10000
N_EDGES = 320000
NUM_BOND_TYPES = 4
BN_EPS = 1e-5

_NW = 32
_EDGE_B = 128
_K = 80
_EP = _NW * _K * _EDGE_B
_NSLOT = 5
_L = 4
_ACC_ROWS = 40448
_PAD_ROWS = _ACC_ROWS - N_NODES * NUM_BOND_TYPES
_ZROWS = 158
_ROWS_PER_TILE = _ACC_ROWS // 16


def _sc_agg(dch: int):
    mesh = plsc.VectorSubcoreMesh(core_axis_name="c", subcore_axis_name="s")

    @functools.partial(
        pl.kernel,
        out_type=jax.ShapeDtypeStruct((2, dch, _ACC_ROWS, 32), jnp.float32),
        mesh=mesh,
        compiler_params=pltpu.CompilerParams(use_tc_tiling_on_sc=False),
        scratch_types=[
            pltpu.VMEM((_K, _EDGE_B), jnp.int32),
            pltpu.VMEM((_K, _EDGE_B), jnp.int32),
            pltpu.VMEM((_NSLOT, _EDGE_B, 32), jnp.float32),
            pltpu.VMEM((_ZROWS, 32), jnp.float32),
            pltpu.VMEM_SHARED((_ACC_ROWS, 32), jnp.float32),
            [pltpu.SemaphoreType.DMA] * _NSLOT,
            [pltpu.SemaphoreType.DMA] * _NSLOT,
            pltpu.SemaphoreType.DMA,
            pltpu.SemaphoreType.DMA,
            pltpu.SemaphoreType.DMA,
        ],
    )
    def agg(xflat, idxp, gidx, zc, out, idx_v, gidx_v, rows_v, zbuf, acc,
            gsems, ssems, zsem, osem, xsem):
        cid = lax.axis_index("c")
        ws = lax.axis_index("s")
        wid = cid * 16 + ws

        def gather(j, s):
            return pltpu.async_copy(xflat.at[gidx_v.at[j]], rows_v.at[s],
                                    gsems[s])

        def gather_wait(j, s):
            pltpu.make_async_copy(xflat.at[gidx_v.at[j]], rows_v.at[s],
                                  gsems[s]).wait()

        def scatter(j, s):
            return pltpu.async_copy(rows_v.at[s], acc.at[idx_v.at[j]],
                                    ssems[s], add=True)

        def scatter_wait(j, s):
            pltpu.make_async_copy(rows_v.at[s], acc.at[idx_v.at[j]],
                                  ssems[s]).wait()

        def issue_zero():
            return [pltpu.async_copy(
                zbuf, acc.at[pl.ds(ws * _ROWS_PER_TILE + q * _ZROWS, _ZROWS)],
                zsem) for q in range(_ROWS_PER_TILE // _ZROWS)]

        pltpu.sync_copy(zc, zbuf)
        pltpu.sync_copy(idxp.at[wid], idx_v)
        pltpu.sync_copy(gidx.at[0, wid], gidx_v)
        zds = issue_zero()
        for c in range(dch):
            for b in range(_L):
                gather(b, b)
            for zd in zds:
                zd.wait()
            plsc.subcore_barrier()

            def stage(j):
                sw = j + _L - _NSLOT
                if sw >= 0:
                    scatter_wait(sw, sw % _NSLOT)
                if j + _L < _K:
                    gather(j + _L, (j + _L) % _NSLOT)
                gather_wait(j, j % _NSLOT)
                scatter(j, j % _NSLOT)

            for j in range(_NSLOT):
                stage(j)

            @pl.loop(1, _K // _NSLOT - 1)
            def _(g):
                for b in range(_NSLOT):
                    j = g * _NSLOT + b
                    s_la = (b + _L) % _NSLOT
                    scatter_wait(j + _L - _NSLOT, s_la)
                    gather(j + _L, s_la)
                    gather_wait(j, b)
                    scatter(j, b)

            for j in range(_K - _NSLOT, _K):
                stage(j)
            for j in range(_K - (_NSLOT - _L), _K):
                scatter_wait(j, j % _NSLOT)

            plsc.subcore_barrier()
            od = pltpu.async_copy(
                acc.at[pl.ds(ws * _ROWS_PER_TILE, _ROWS_PER_TILE)],
                out.at[cid, c, pl.ds(ws * _ROWS_PER_TILE, _ROWS_PER_TILE), :],
                osem)
            if c + 1 < dch:
                gl = pltpu.async_copy(gidx.at[c + 1, wid], gidx_v, xsem)
                od.wait()
                zds = issue_zero()
                gl.wait()
            else:
                od.wait()

    return agg


def _elu(z):
    return jnp.where(z > 0, z, jnp.exp(z) - 1.0)


_RB = 1000
_PREC = lax.Precision.DEFAULT


def _bn_linear(xs, gamma, beta, w):
    n = xs[0].shape[0]
    h = w.shape[1]
    fins = [x.shape[1] for x in xs]
    offs = [0]
    for f in fins:
        offs.append(offs[-1] + f)
    scale = gamma / jnp.sqrt(jnp.float32(1.0 + BN_EPS))
    sps = [scale[offs[k]:offs[k + 1]].reshape(1, fins[k]) for k in range(len(xs))]
    bps = [beta[offs[k]:offs[k + 1]].reshape(1, fins[k]) for k in range(len(xs))]
    wps = [w[offs[k]:offs[k + 1]] for k in range(len(xs))]

    def body(*refs):
        nseg = len(fins)
        x_refs = refs[:nseg]
        s_refs = refs[nseg:2 * nseg]
        b_refs = refs[2 * nseg:3 * nseg]
        w_refs = refs[3 * nseg:4 * nseg]
        o_ref = refs[4 * nseg]
        acc = None
        for k in range(nseg):
            z = x_refs[k][...] * s_refs[k][...] + b_refs[k][...]
            part = jnp.dot(_elu(z), w_refs[k][...],
                           preferred_element_type=jnp.float32, precision=_PREC)
            acc = part if acc is None else acc + part
        o_ref[...] = acc

    return pl.pallas_call(
        body,
        grid=(n // _RB,),
        in_specs=(
            [pl.BlockSpec((_RB, f), lambda i: (i, 0)) for f in fins]
            + [pl.BlockSpec((1, f), lambda i: (0, 0)) for f in fins]
            + [pl.BlockSpec((1, f), lambda i: (0, 0)) for f in fins]
            + [pl.BlockSpec((f, h), lambda i: (0, 0)) for f in fins]
        ),
        out_specs=pl.BlockSpec((_RB, h), lambda i: (i, 0)),
        out_shape=jax.ShapeDtypeStruct((n, h), jnp.float32),
    )(*xs, *sps, *bps, *wps)


def _molconv_dense(x, aggp, gamma, beta, w):
    n, d = x.shape
    h = w.shape[1]
    scale = gamma / jnp.sqrt(jnp.float32(1.0 + BN_EPS))
    sp = jnp.concatenate([scale[:d].reshape(1, 1, d),
                          scale[d:].reshape(4, 1, d)], axis=0)
    bp = jnp.concatenate([beta[:d].reshape(1, 1, d),
                          beta[d:].reshape(4, 1, d)], axis=0)
    wp = jnp.concatenate([w[:d].reshape(1, d, h),
                          w[d:].reshape(4, d, h)], axis=0)

    dch = d // 32

    def body(x_ref, a_ref, s_ref, b_ref, w_ref, o_ref):
        z = x_ref[...] * s_ref[0] + b_ref[0]
        acc = jnp.dot(_elu(z), w_ref[0], preferred_element_type=jnp.float32,
                      precision=_PREC)
        a = a_ref[0] + a_ref[1]
        for t in range(4):
            zt = jnp.concatenate(
                [a[ch, :, t * 32:(t + 1) * 32] for ch in range(dch)], axis=-1)
            z = zt * s_ref[t + 1] + b_ref[t + 1]
            acc += jnp.dot(_elu(z), w_ref[t + 1],
                           preferred_element_type=jnp.float32, precision=_PREC)
        o_ref[...] = acc

    return pl.pallas_call(
        body,
        grid=(n // _RB,),
        in_specs=[
            pl.BlockSpec((_RB, d), lambda i: (i, 0)),
            pl.BlockSpec((2, dch, _RB, 128), lambda i: (0, 0, i, 0)),
            pl.BlockSpec((5, 1, d), lambda i: (0, 0, 0)),
            pl.BlockSpec((5, 1, d), lambda i: (0, 0, 0)),
            pl.BlockSpec((5, d, h), lambda i: (0, 0, 0)),
        ],
        out_specs=pl.BlockSpec((_RB, h), lambda i: (i, 0)),
        out_shape=jax.ShapeDtypeStruct((n, h), jnp.float32),
    )(x, aggp, sp, bp, wp)


def _edge_plans(bond_info):
    begin = bond_info[:, 0]
    end = bond_info[:, 1]
    bt = bond_info[:, 2] % NUM_BOND_TYPES
    idx = end * NUM_BOND_TYPES + bt
    pad = _EP - N_EDGES
    pr = jnp.arange(pad, dtype=jnp.int32)
    idxp = jnp.concatenate(
        [idx, N_NODES * NUM_BOND_TYPES + pr % _PAD_ROWS]).reshape(_NW, _K, _EDGE_B)
    beginp = jnp.concatenate([begin, pr % N_NODES])
    gidx = {}
    for dch in (4, 8):
        gidx[dch] = (
            beginp[None, :] * dch + jnp.arange(dch, dtype=jnp.int32)[:, None]
        ).reshape(dch, _NW, _K, _EDGE_B)
    zc = jnp.zeros((_ZROWS, 32), jnp.float32)
    return idxp, gidx, zc


def kernel(atom_features, bond_info, params):
    idxp, gidx, zc = _edge_plans(bond_info)

    def molconv(x, gamma, beta, w):
        d = x.shape[1]
        dch = d // 32
        xflat = x.reshape(N_NODES * dch, 32)
        aggp = _sc_agg(dch)(xflat, idxp, gidx[dch], zc)
        aggp = aggp.reshape(2, dch, _ACC_ROWS // 4, 128)
        return _molconv_dense(x, aggp, gamma, beta, w)

    x = atom_features
    for i in range(2):
        x = molconv(x, params['causal%d_gamma' % i], params['causal%d_beta' % i],
                    params['causal%d_W' % i])
    feats = [x]
    for i in range(3):
        b = _bn_linear(feats, params['dense%d_bn_gamma' % i],
                       params['dense%d_bn_beta' % i], params['dense%d_bn_W' % i])
        y = molconv(b, params['dense%d_conv_gamma' % i],
                    params['dense%d_conv_beta' % i], params['dense%d_conv_W' % i])
        feats.append(y)
    return _bn_linear(feats, params['out_gamma'], params['out_beta'],
                      params['out_W'])


# speedup vs baseline: 8.0427x; 1.0033x over previous
import functools

import jax
import jax.numpy as jnp
from jax import lax
from jax.experimental import pallas as pl
from jax.experimental.pallas import tpu as pltpu
from jax.experimental.pallas import tpu_sc as plsc

N_NODES = 10000
N_EDGES = 320000
NUM_BOND_TYPES = 4
BN_EPS = 1e-5

_NW = 32
_EDGE_B = 128
_K = 80
_EP = _NW * _K * _EDGE_B
_NSLOT = 5
_L = 4
_ACC_ROWS = 40448
_PAD_ROWS = _ACC_ROWS - N_NODES * NUM_BOND_TYPES
_ZROWS = 158
_ROWS_PER_TILE = _ACC_ROWS // 16


def _sc_agg(dch: int):
    mesh = plsc.VectorSubcoreMesh(core_axis_name="c", subcore_axis_name="s")

    @functools.partial(
        pl.kernel,
        out_type=jax.ShapeDtypeStruct((2, dch, _ACC_ROWS, 32), jnp.float32),
        mesh=mesh,
        compiler_params=pltpu.CompilerParams(use_tc_tiling_on_sc=False),
        scratch_types=[
            pltpu.VMEM((_K, _EDGE_B), jnp.int32),
            pltpu.VMEM((_K, _EDGE_B), jnp.int32),
            pltpu.VMEM((_NSLOT, _EDGE_B, 32), jnp.float32),
            pltpu.VMEM((_ZROWS, 32), jnp.float32),
            pltpu.VMEM_SHARED((_ACC_ROWS, 32), jnp.float32),
            [pltpu.SemaphoreType.DMA] * _NSLOT,
            [pltpu.SemaphoreType.DMA] * _NSLOT,
            pltpu.SemaphoreType.DMA,
            pltpu.SemaphoreType.DMA,
            pltpu.SemaphoreType.DMA,
        ],
    )
    def agg(xflat, idxp, gidx, zc, out, idx_v, gidx_v, rows_v, zbuf, acc,
            gsems, ssems, zsem, osem, xsem):
        cid = lax.axis_index("c")
        ws = lax.axis_index("s")
        wid = cid * 16 + ws

        def gather(j, s):
            return pltpu.async_copy(xflat.at[gidx_v.at[j]], rows_v.at[s],
                                    gsems[s])

        def gather_wait(j, s):
            pltpu.make_async_copy(xflat.at[gidx_v.at[j]], rows_v.at[s],
                                  gsems[s]).wait()

        def scatter(j, s):
            return pltpu.async_copy(rows_v.at[s], acc.at[idx_v.at[j]],
                                    ssems[s], add=True)

        def scatter_wait(j, s):
            pltpu.make_async_copy(rows_v.at[s], acc.at[idx_v.at[j]],
                                  ssems[s]).wait()

        def issue_zero():
            return [pltpu.async_copy(
                zbuf, acc.at[pl.ds(ws * _ROWS_PER_TILE + q * _ZROWS, _ZROWS)],
                zsem) for q in range(_ROWS_PER_TILE // _ZROWS)]

        pltpu.sync_copy(zc, zbuf)
        pltpu.sync_copy(idxp.at[wid], idx_v)
        pltpu.sync_copy(gidx.at[0, wid], gidx_v)
        zds = issue_zero()
        for c in range(dch):
            for b in range(_L):
                gather(b, b)
            for zd in zds:
                zd.wait()
            plsc.subcore_barrier()

            def stage(j):
                sw = j + _L - _NSLOT
                if sw >= 0:
                    scatter_wait(sw, sw % _NSLOT)
                if j + _L < _K:
                    gather(j + _L, (j + _L) % _NSLOT)
                gather_wait(j, j % _NSLOT)
                scatter(j, j % _NSLOT)

            for j in range(_NSLOT):
                stage(j)

            @pl.loop(1, _K // _NSLOT - 1)
            def _(g):
                for b in range(_NSLOT):
                    j = g * _NSLOT + b
                    s_la = (b + _L) % _NSLOT
                    scatter_wait(j + _L - _NSLOT, s_la)
                    gather(j + _L, s_la)
                    gather_wait(j, b)
                    scatter(j, b)

            for j in range(_K - _NSLOT, _K):
                stage(j)
            for j in range(_K - (_NSLOT - _L), _K):
                scatter_wait(j, j % _NSLOT)

            plsc.subcore_barrier()
            od = pltpu.async_copy(
                acc.at[pl.ds(ws * _ROWS_PER_TILE, _ROWS_PER_TILE)],
                out.at[cid, c, pl.ds(ws * _ROWS_PER_TILE, _ROWS_PER_TILE), :],
                osem)
            if c + 1 < dch:
                gl = pltpu.async_copy(gidx.at[c + 1, wid], gidx_v, xsem)
                od.wait()
                zds = issue_zero()
                gl.wait()
            else:
                od.wait()

    return agg


def _elu(z):
    return jnp.where(z > 0, z, jnp.exp(z) - 1.0)


_RB = 2000
_PREC = lax.Precision.DEFAULT


def _bn_linear(xs, gamma, beta, w):
    n = xs[0].shape[0]
    h = w.shape[1]
    fins = [x.shape[1] for x in xs]
    offs = [0]
    for f in fins:
        offs.append(offs[-1] + f)
    scale = gamma / jnp.sqrt(jnp.float32(1.0 + BN_EPS))
    sps = [scale[offs[k]:offs[k + 1]].reshape(1, fins[k]) for k in range(len(xs))]
    bps = [beta[offs[k]:offs[k + 1]].reshape(1, fins[k]) for k in range(len(xs))]
    wps = [w[offs[k]:offs[k + 1]] for k in range(len(xs))]

    def body(*refs):
        nseg = len(fins)
        x_refs = refs[:nseg]
        s_refs = refs[nseg:2 * nseg]
        b_refs = refs[2 * nseg:3 * nseg]
        w_refs = refs[3 * nseg:4 * nseg]
        o_ref = refs[4 * nseg]
        acc = None
        for k in range(nseg):
            z = x_refs[k][...] * s_refs[k][...] + b_refs[k][...]
            part = jnp.dot(_elu(z), w_refs[k][...],
                           preferred_element_type=jnp.float32, precision=_PREC)
            acc = part if acc is None else acc + part
        o_ref[...] = acc

    return pl.pallas_call(
        body,
        grid=(n // _RB,),
        in_specs=(
            [pl.BlockSpec((_RB, f), lambda i: (i, 0)) for f in fins]
            + [pl.BlockSpec((1, f), lambda i: (0, 0)) for f in fins]
            + [pl.BlockSpec((1, f), lambda i: (0, 0)) for f in fins]
            + [pl.BlockSpec((f, h), lambda i: (0, 0)) for f in fins]
        ),
        out_specs=pl.BlockSpec((_RB, h), lambda i: (i, 0)),
        out_shape=jax.ShapeDtypeStruct((n, h), jnp.float32),
    )(*xs, *sps, *bps, *wps)


def _molconv_dense(x, aggp, gamma, beta, w):
    n, d = x.shape
    h = w.shape[1]
    scale = gamma / jnp.sqrt(jnp.float32(1.0 + BN_EPS))
    sp = jnp.concatenate([scale[:d].reshape(1, 1, d),
                          scale[d:].reshape(4, 1, d)], axis=0)
    bp = jnp.concatenate([beta[:d].reshape(1, 1, d),
                          beta[d:].reshape(4, 1, d)], axis=0)
    wp = jnp.concatenate([w[:d].reshape(1, d, h),
                          w[d:].reshape(4, d, h)], axis=0)

    dch = d // 32

    def body(x_ref, a_ref, s_ref, b_ref, w_ref, o_ref):
        z = x_ref[...] * s_ref[0] + b_ref[0]
        acc = jnp.dot(_elu(z), w_ref[0], preferred_element_type=jnp.float32,
                      precision=_PREC)
        a = a_ref[0] + a_ref[1]
        for t in range(4):
            zt = jnp.concatenate(
                [a[ch, :, t * 32:(t + 1) * 32] for ch in range(dch)], axis=-1)
            z = zt * s_ref[t + 1] + b_ref[t + 1]
            acc += jnp.dot(_elu(z), w_ref[t + 1],
                           preferred_element_type=jnp.float32, precision=_PREC)
        o_ref[...] = acc

    return pl.pallas_call(
        body,
        grid=(n // _RB,),
        in_specs=[
            pl.BlockSpec((_RB, d), lambda i: (i, 0)),
            pl.BlockSpec((2, dch, _RB, 128), lambda i: (0, 0, i, 0)),
            pl.BlockSpec((5, 1, d), lambda i: (0, 0, 0)),
            pl.BlockSpec((5, 1, d), lambda i: (0, 0, 0)),
            pl.BlockSpec((5, d, h), lambda i: (0, 0, 0)),
        ],
        out_specs=pl.BlockSpec((_RB, h), lambda i: (i, 0)),
        out_shape=jax.ShapeDtypeStruct((n, h), jnp.float32),
    )(x, aggp, sp, bp, wp)


def _edge_plans(bond_info):
    begin = bond_info[:, 0]
    end = bond_info[:, 1]
    bt = bond_info[:, 2] % NUM_BOND_TYPES
    idx = end * NUM_BOND_TYPES + bt
    pad = _EP - N_EDGES
    pr = jnp.arange(pad, dtype=jnp.int32)
    idxp = jnp.concatenate(
        [idx, N_NODES * NUM_BOND_TYPES + pr % _PAD_ROWS]).reshape(_NW, _K, _EDGE_B)
    beginp = jnp.concatenate([begin, pr % N_NODES])
    gidx = {}
    for dch in (4, 8):
        gidx[dch] = (
            beginp[None, :] * dch + jnp.arange(dch, dtype=jnp.int32)[:, None]
        ).reshape(dch, _NW, _K, _EDGE_B)
    zc = jnp.zeros((_ZROWS, 32), jnp.float32)
    return idxp, gidx, zc


def kernel(atom_features, bond_info, params):
    idxp, gidx, zc = _edge_plans(bond_info)

    def molconv(x, gamma, beta, w):
        d = x.shape[1]
        dch = d // 32
        xflat = x.reshape(N_NODES * dch, 32)
        aggp = _sc_agg(dch)(xflat, idxp, gidx[dch], zc)
        aggp = aggp.reshape(2, dch, _ACC_ROWS // 4, 128)
        return _molconv_dense(x, aggp, gamma, beta, w)

    x = atom_features
    for i in range(2):
        x = molconv(x, params['causal%d_gamma' % i], params['causal%d_beta' % i],
                    params['causal%d_W' % i])
    feats = [x]
    for i in range(3):
        b = _bn_linear(feats, params['dense%d_bn_gamma' % i],
                       params['dense%d_bn_beta' % i], params['dense%d_bn_W' % i])
        y = molconv(b, params['dense%d_conv_gamma' % i],
                    params['dense%d_conv_beta' % i], params['dense%d_conv_W' % i])
        feats.append(y)
    return _bn_linear(feats, params['out_gamma'], params['out_beta'],
                      params['out_W'])
